# baseline (device time: 69586 ns/iter reference)
import jax
import jax.numpy as jnp
from jax import lax
from jax.experimental import pallas as pl
from jax.experimental.pallas import tpu as pltpu

N_DEV = 32
LOG2_N = 5
B, Sq, D = 2, 256, 768
Hq, Dh = 8, 64
HD = Hq * Dh
BH = B * Hq
S = 4
SW = Sq // S
FH = Sq // 2
R = Dh + 2


def kernel(x, Wq, Wo, K_ext, V_ext):
    skv_loc = K_ext.shape[1]

    def body(x_ref, wq_ref, wo_ref, k_ref, v_ref, out_ref,
             acc_ref, rx_ref, send_sems, recv_sems):
        my = lax.axis_index("i")

        x2 = x_ref[...].reshape(B * Sq, D).astype(jnp.bfloat16)
        wq = wq_ref[...].astype(jnp.bfloat16)
        qT = lax.dot_general(wq, x2, (((0,), (1,)), ((), ())),
                             preferred_element_type=jnp.float32)
        qT = qT * 0.125

        kbs = []
        vbs = []
        for b in range(B):
            kbs.append(k_ref[b, :, :, :].reshape(skv_loc, HD).astype(jnp.bfloat16))
            vbs.append(v_ref[b, :, :, :].reshape(skv_loc, HD).astype(jnp.bfloat16))

        def flash_round(half):
            for b in range(B):
                for h in range(Hq):
                    bh = b * Hq + h
                    c0 = b * Sq + half * FH
                    q_bh = qT[h * Dh:(h + 1) * Dh,
                              c0:c0 + FH].astype(jnp.bfloat16)
                    k_bh = kbs[b][:, h * Dh:(h + 1) * Dh]
                    v_bh = vbs[b][:, h * Dh:(h + 1) * Dh]
                    sT = lax.dot_general(k_bh, q_bh, (((1,), (0,)), ((), ())),
                                         preferred_element_type=jnp.float32)
                    m = jnp.max(sT, axis=0, keepdims=True)
                    p = jnp.exp(sT - m)
                    l = jnp.sum(p, axis=0, keepdims=True)
                    oT = lax.dot_general(v_bh, p.astype(jnp.bfloat16),
                                         (((0,), (0,)), ((), ())),
                                         preferred_element_type=jnp.float32)
                    ob = oT.astype(jnp.bfloat16)
                    mb = m.astype(jnp.bfloat16)
                    lb = l.astype(jnp.bfloat16)
                    for q in range(2):
                        j = 2 * half + q
                        cs = slice(q * SW, (q + 1) * SW)
                        acc_ref[j, bh, 0:Dh, :] = ob[:, cs]
                        acc_ref[j, bh, Dh:Dh + 1, :] = mb[:, cs]
                        acc_ref[j, bh, Dh + 1:Dh + 2, :] = lb[:, cs]

        def start_exchange(j, step):
            partner = my ^ (1 << ((step + j) % LOG2_N))
            rd = pltpu.make_async_remote_copy(
                src_ref=acc_ref.at[j],
                dst_ref=rx_ref.at[step, j],
                send_sem=send_sems.at[step, j],
                recv_sem=recv_sems.at[step, j],
                device_id=(partner,),
                device_id_type=pl.DeviceIdType.MESH,
            )
            rd.start()
            return rd

        def combine(j, step):
            m1 = acc_ref[j, :, Dh:Dh + 1, :].astype(jnp.float32)
            l1 = acc_ref[j, :, Dh + 1:Dh + 2, :].astype(jnp.float32)
            m2 = rx_ref[step, j, :, Dh:Dh + 1, :].astype(jnp.float32)
            l2 = rx_ref[step, j, :, Dh + 1:Dh + 2, :].astype(jnp.float32)
            mn = jnp.maximum(m1, m2)
            a1 = jnp.exp(m1 - mn)
            a2 = jnp.exp(m2 - mn)
            o_new = (a1 * acc_ref[j, :, 0:Dh, :].astype(jnp.float32)
                     + a2 * rx_ref[step, j, :, 0:Dh, :].astype(jnp.float32))
            acc_ref[j, :, 0:Dh, :] = o_new.astype(jnp.bfloat16)
            acc_ref[j, :, Dh:Dh + 1, :] = mn.astype(jnp.bfloat16)
            acc_ref[j, :, Dh + 1:Dh + 2, :] = (a1 * l1 + a2 * l2).astype(jnp.bfloat16)

        wo = wo_ref[...].astype(jnp.bfloat16)

        def project_half(half):
            j0, j1 = 2 * half, 2 * half + 1
            linv = 1.0 / jnp.concatenate(
                [acc_ref[j0, :, Dh + 1:Dh + 2, :].astype(jnp.float32),
                 acc_ref[j1, :, Dh + 1:Dh + 2, :].astype(jnp.float32)],
                axis=-1)
            o_cat = jnp.concatenate(
                [acc_ref[j0, :, 0:Dh, :].astype(jnp.float32),
                 acc_ref[j1, :, 0:Dh, :].astype(jnp.float32)],
                axis=-1)
            o_n = (o_cat * linv).astype(jnp.bfloat16)
            r0 = half * FH
            for b in range(B):
                acc = jnp.zeros((FH, D), jnp.float32)
                for h in range(Hq):
                    bh = b * Hq + h
                    wo_h = wo[h * Dh:(h + 1) * Dh, :]
                    acc = acc + lax.dot_general(
                        o_n[bh], wo_h, (((0,), (0,)), ((), ())),
                        preferred_element_type=jnp.float32)
                out_ref[b, r0:r0 + FH, :] = acc

        rds = [None] * S
        flash_round(0)
        rds[0] = start_exchange(0, 0)
        rds[1] = start_exchange(1, 0)
        flash_round(1)
        rds[2] = start_exchange(2, 0)
        rds[3] = start_exchange(3, 0)
        for step in range(LOG2_N):
            for j in range(S):
                rds[j].wait()
                combine(j, step)
                if step + 1 < LOG2_N:
                    rds[j] = start_exchange(j, step + 1)
                elif j % 2 == 1:
                    project_half(j // 2)

    return pl.pallas_call(
        body,
        out_shape=jax.ShapeDtypeStruct((B, Sq, D), jnp.float32),
        in_specs=[pl.BlockSpec(memory_space=pltpu.VMEM)] * 5,
        out_specs=pl.BlockSpec(memory_space=pltpu.VMEM),
        scratch_shapes=[
            pltpu.VMEM((S, BH, R, SW), jnp.bfloat16),
            pltpu.VMEM((LOG2_N, S, BH, R, SW), jnp.bfloat16),
            pltpu.SemaphoreType.DMA((LOG2_N, S)),
            pltpu.SemaphoreType.DMA((LOG2_N, S)),
        ],
    )(x, Wq, Wo, K_ext, V_ext)


# device time: 64106 ns/iter; 1.0855x vs baseline; 1.0855x over previous
import jax
import jax.numpy as jnp
from jax import lax
from jax.experimental import pallas as pl
from jax.experimental.pallas import tpu as pltpu

N_DEV = 32
LOG2_N = 5
B, Sq, D = 2, 256, 768
Hq, Dh = 8, 64
HD = Hq * Dh
BH = B * Hq
SH = Sq // 2

BITS_A = (0, 1, 2, 3, 4)
BITS_B = (2, 3, 4, 0, 1)


def kernel(x, Wq, Wo, K_ext, V_ext):
    skv_loc = K_ext.shape[1]

    def body(x_ref, wq_ref, wo_ref, k_ref, v_ref, out_ref,
             o_acc, ml_acc, o_rx, ml_rx,
             o_send_sems, o_recv_sems, ml_send_sems, ml_recv_sems):
        my = lax.axis_index("i")

        x2 = x_ref[...].reshape(B * Sq, D).astype(jnp.bfloat16)
        wq = wq_ref[...].astype(jnp.bfloat16)
        qT = lax.dot_general(wq, x2, (((0,), (1,)), ((), ())),
                             preferred_element_type=jnp.float32)
        qT = qT * 0.125

        kbs = []
        vbs = []
        for b in range(B):
            kbs.append(k_ref[b, :, :, :].reshape(skv_loc, HD).astype(jnp.bfloat16))
            vbs.append(v_ref[b, :, :, :].reshape(skv_loc, HD).astype(jnp.bfloat16))

        def flash_half(half):
            for b in range(B):
                for h in range(Hq):
                    bh = b * Hq + h
                    c0 = b * Sq + half * SH
                    q_bh = qT[h * Dh:(h + 1) * Dh,
                              c0:c0 + SH].astype(jnp.bfloat16)
                    k_bh = kbs[b][:, h * Dh:(h + 1) * Dh]
                    v_bh = vbs[b][:, h * Dh:(h + 1) * Dh]
                    sT = lax.dot_general(k_bh, q_bh, (((1,), (0,)), ((), ())),
                                         preferred_element_type=jnp.float32)
                    m = jnp.max(sT, axis=0, keepdims=True)
                    p = jnp.exp(sT - m)
                    l = jnp.sum(p, axis=0, keepdims=True)
                    oT = lax.dot_general(v_bh, p.astype(jnp.bfloat16),
                                         (((0,), (0,)), ((), ())),
                                         preferred_element_type=jnp.float32)
                    o_acc[half, bh, :, :] = oT.astype(jnp.bfloat16)
                    ml_acc[half, bh, 0:1, :] = m
                    ml_acc[half, bh, 1:2, :] = l

        def start_exchange(half, step):
            bit = (BITS_A, BITS_B)[half][step]
            partner = my ^ (1 << bit)
            o_rd = pltpu.make_async_remote_copy(
                src_ref=o_acc.at[half],
                dst_ref=o_rx.at[step, half],
                send_sem=o_send_sems.at[step, half],
                recv_sem=o_recv_sems.at[step, half],
                device_id=(partner,),
                device_id_type=pl.DeviceIdType.MESH,
            )
            ml_rd = pltpu.make_async_remote_copy(
                src_ref=ml_acc.at[half],
                dst_ref=ml_rx.at[step, half],
                send_sem=ml_send_sems.at[step, half],
                recv_sem=ml_recv_sems.at[step, half],
                device_id=(partner,),
                device_id_type=pl.DeviceIdType.MESH,
            )
            o_rd.start()
            ml_rd.start()
            return o_rd, ml_rd

        def combine(half, step):
            m1 = ml_acc[half, :, 0:1, :]
            l1 = ml_acc[half, :, 1:2, :]
            m2 = ml_rx[step, half, :, 0:1, :]
            l2 = ml_rx[step, half, :, 1:2, :]
            mn = jnp.maximum(m1, m2)
            a1 = jnp.exp(m1 - mn)
            a2 = jnp.exp(m2 - mn)
            ml_acc[half, :, 0:1, :] = mn
            ml_acc[half, :, 1:2, :] = a1 * l1 + a2 * l2
            o_new = (a1 * o_acc[half].astype(jnp.float32)
                     + a2 * o_rx[step, half].astype(jnp.float32))
            o_acc[half] = o_new.astype(jnp.bfloat16)

        wo = wo_ref[...].astype(jnp.bfloat16)

        def project(half):
            linv = 1.0 / ml_acc[half, :, 1:2, :]
            r0 = half * SH
            for b in range(B):
                acc = jnp.zeros((SH, D), jnp.float32)
                for h in range(Hq):
                    bh = b * Hq + h
                    o_n = (o_acc[half, bh, :, :].astype(jnp.float32)
                           * linv[bh, :, :]).astype(jnp.bfloat16)
                    wo_h = wo[h * Dh:(h + 1) * Dh, :]
                    acc = acc + lax.dot_general(
                        o_n, wo_h, (((0,), (0,)), ((), ())),
                        preferred_element_type=jnp.float32)
                out_ref[b, r0:r0 + SH, :] = acc

        flash_half(0)
        rd0 = start_exchange(0, 0)
        flash_half(1)
        rd1 = start_exchange(1, 0)
        for step in range(LOG2_N):
            rd0[0].wait()
            rd0[1].wait()
            combine(0, step)
            if step + 1 < LOG2_N:
                rd0 = start_exchange(0, step + 1)
            else:
                project(0)
            rd1[0].wait()
            rd1[1].wait()
            combine(1, step)
            if step + 1 < LOG2_N:
                rd1 = start_exchange(1, step + 1)
            else:
                project(1)

    return pl.pallas_call(
        body,
        out_shape=jax.ShapeDtypeStruct((B, Sq, D), jnp.float32),
        in_specs=[pl.BlockSpec(memory_space=pltpu.VMEM)] * 5,
        out_specs=pl.BlockSpec(memory_space=pltpu.VMEM),
        scratch_shapes=[
            pltpu.VMEM((2, BH, Dh, SH), jnp.bfloat16),
            pltpu.VMEM((2, BH, 2, SH), jnp.float32),
            pltpu.VMEM((LOG2_N, 2, BH, Dh, SH), jnp.bfloat16),
            pltpu.VMEM((LOG2_N, 2, BH, 2, SH), jnp.float32),
            pltpu.SemaphoreType.DMA((LOG2_N, 2)),
            pltpu.SemaphoreType.DMA((LOG2_N, 2)),
            pltpu.SemaphoreType.DMA((LOG2_N, 2)),
            pltpu.SemaphoreType.DMA((LOG2_N, 2)),
        ],
    )(x, Wq, Wo, K_ext, V_ext)


# device time: 62450 ns/iter; 1.1143x vs baseline; 1.0265x over previous
import jax
import jax.numpy as jnp
from jax import lax
from jax.experimental import pallas as pl
from jax.experimental.pallas import tpu as pltpu

N_DEV = 32
LOG2_N = 5
B, Sq, D = 2, 256, 768
Hq, Dh = 8, 64
HD = Hq * Dh
BH = B * Hq
SH = Sq // 2

BITS_A = (2, 4, 0, 1, 3)
BITS_B = (4, 2, 1, 3, 0)


def kernel(x, Wq, Wo, K_ext, V_ext):
    skv_loc = K_ext.shape[1]

    def body(x_ref, wq_ref, wo_ref, k_ref, v_ref, out_ref,
             o_acc, ml_acc, o_rx, ml_rx,
             o_send_sems, o_recv_sems, ml_send_sems, ml_recv_sems):
        my = lax.axis_index("i")

        x2 = x_ref[...].reshape(B * Sq, D).astype(jnp.bfloat16)
        wq = wq_ref[...].astype(jnp.bfloat16)
        qT = lax.dot_general(wq, x2, (((0,), (1,)), ((), ())),
                             preferred_element_type=jnp.float32)
        qT = qT * 0.125

        kbs = []
        vbs = []
        for b in range(B):
            kbs.append(k_ref[b, :, :, :].reshape(skv_loc, HD).astype(jnp.bfloat16))
            vbs.append(v_ref[b, :, :, :].reshape(skv_loc, HD).astype(jnp.bfloat16))

        def flash_half(half):
            for b in range(B):
                for h in range(Hq):
                    bh = b * Hq + h
                    c0 = b * Sq + half * SH
                    q_bh = qT[h * Dh:(h + 1) * Dh,
                              c0:c0 + SH].astype(jnp.bfloat16)
                    k_bh = kbs[b][:, h * Dh:(h + 1) * Dh]
                    v_bh = vbs[b][:, h * Dh:(h + 1) * Dh]
                    sT = lax.dot_general(k_bh, q_bh, (((1,), (0,)), ((), ())),
                                         preferred_element_type=jnp.float32)
                    m = jnp.max(sT, axis=0, keepdims=True)
                    p = jnp.exp(sT - m)
                    l = jnp.sum(p, axis=0, keepdims=True)
                    oT = lax.dot_general(v_bh, p.astype(jnp.bfloat16),
                                         (((0,), (0,)), ((), ())),
                                         preferred_element_type=jnp.float32)
                    o_acc[half, bh, :, :] = oT.astype(jnp.bfloat16)
                    ml_acc[half, bh, 0:1, :] = m
                    ml_acc[half, bh, 1:2, :] = l

        def start_exchange(half, step):
            bit = (BITS_A, BITS_B)[half][step]
            partner = my ^ (1 << bit)
            o_rd = pltpu.make_async_remote_copy(
                src_ref=o_acc.at[half],
                dst_ref=o_rx.at[step, half],
                send_sem=o_send_sems.at[step, half],
                recv_sem=o_recv_sems.at[step, half],
                device_id=(partner,),
                device_id_type=pl.DeviceIdType.MESH,
            )
            ml_rd = pltpu.make_async_remote_copy(
                src_ref=ml_acc.at[half],
                dst_ref=ml_rx.at[step, half],
                send_sem=ml_send_sems.at[step, half],
                recv_sem=ml_recv_sems.at[step, half],
                device_id=(partner,),
                device_id_type=pl.DeviceIdType.MESH,
            )
            o_rd.start()
            ml_rd.start()
            return o_rd, ml_rd

        def combine(half, step):
            m1 = ml_acc[half, :, 0:1, :]
            l1 = ml_acc[half, :, 1:2, :]
            m2 = ml_rx[step, half, :, 0:1, :]
            l2 = ml_rx[step, half, :, 1:2, :]
            mn = jnp.maximum(m1, m2)
            a1 = jnp.exp(m1 - mn)
            a2 = jnp.exp(m2 - mn)
            ml_acc[half, :, 0:1, :] = mn
            ml_acc[half, :, 1:2, :] = a1 * l1 + a2 * l2
            o_new = (a1 * o_acc[half].astype(jnp.float32)
                     + a2 * o_rx[step, half].astype(jnp.float32))
            o_acc[half] = o_new.astype(jnp.bfloat16)

        wo = wo_ref[...].astype(jnp.bfloat16)

        def project(half):
            linv = 1.0 / ml_acc[half, :, 1:2, :]
            r0 = half * SH
            for b in range(B):
                acc = jnp.zeros((SH, D), jnp.float32)
                for h in range(Hq):
                    bh = b * Hq + h
                    o_n = (o_acc[half, bh, :, :].astype(jnp.float32)
                           * linv[bh, :, :]).astype(jnp.bfloat16)
                    wo_h = wo[h * Dh:(h + 1) * Dh, :]
                    acc = acc + lax.dot_general(
                        o_n, wo_h, (((0,), (0,)), ((), ())),
                        preferred_element_type=jnp.float32)
                out_ref[b, r0:r0 + SH, :] = acc

        flash_half(0)
        rd0 = start_exchange(0, 0)
        flash_half(1)
        rd1 = start_exchange(1, 0)
        for step in range(LOG2_N):
            rd0[0].wait()
            rd0[1].wait()
            combine(0, step)
            if step + 1 < LOG2_N:
                rd0 = start_exchange(0, step + 1)
            else:
                project(0)
            rd1[0].wait()
            rd1[1].wait()
            combine(1, step)
            if step + 1 < LOG2_N:
                rd1 = start_exchange(1, step + 1)
            else:
                project(1)

    return pl.pallas_call(
        body,
        out_shape=jax.ShapeDtypeStruct((B, Sq, D), jnp.float32),
        in_specs=[pl.BlockSpec(memory_space=pltpu.VMEM)] * 5,
        out_specs=pl.BlockSpec(memory_space=pltpu.VMEM),
        scratch_shapes=[
            pltpu.VMEM((2, BH, Dh, SH), jnp.bfloat16),
            pltpu.VMEM((2, BH, 2, SH), jnp.float32),
            pltpu.VMEM((LOG2_N, 2, BH, Dh, SH), jnp.bfloat16),
            pltpu.VMEM((LOG2_N, 2, BH, 2, SH), jnp.float32),
            pltpu.SemaphoreType.DMA((LOG2_N, 2)),
            pltpu.SemaphoreType.DMA((LOG2_N, 2)),
            pltpu.SemaphoreType.DMA((LOG2_N, 2)),
            pltpu.SemaphoreType.DMA((LOG2_N, 2)),
        ],
    )(x, Wq, Wo, K_ext, V_ext)


# device time: 51212 ns/iter; 1.3588x vs baseline; 1.2194x over previous
import jax
import jax.numpy as jnp
from jax import lax
from jax.experimental import pallas as pl
from jax.experimental.pallas import tpu as pltpu

N_DEV = 32
LOG2_N = 5
B, Sq, D = 2, 256, 768
Hq, Dh = 8, 64
HD = Hq * Dh
BH = B * Hq
S = 4
SB = BH // S


def kernel(x, Wq, Wo, K_ext, V_ext):
    skv_loc = K_ext.shape[1]

    def body(x_ref, wq_ref, wo_ref, k_ref, v_ref, out_ref,
             o_acc, ml_acc, o_rx, ml_rx,
             o_send_sems, o_recv_sems, ml_send_sems, ml_recv_sems):
        my = lax.axis_index("i")

        x2 = x_ref[...].reshape(B * Sq, D).astype(jnp.bfloat16)
        wq = wq_ref[...].astype(jnp.bfloat16)
        qT = lax.dot_general(wq, x2, (((0,), (1,)), ((), ())),
                             preferred_element_type=jnp.float32)
        qT = qT * 0.125

        kbs = []
        vbs = []
        for b in range(B):
            kbs.append(k_ref[b, :, :, :].reshape(skv_loc, HD).astype(jnp.bfloat16))
            vbs.append(v_ref[b, :, :, :].reshape(skv_loc, HD).astype(jnp.bfloat16))

        def flash_strip(j):
            for i in range(SB):
                bh = j * SB + i
                b, h = divmod(bh, Hq)
                q_bh = qT[h * Dh:(h + 1) * Dh,
                          b * Sq:(b + 1) * Sq].astype(jnp.bfloat16)
                k_bh = kbs[b][:, h * Dh:(h + 1) * Dh]
                v_bh = vbs[b][:, h * Dh:(h + 1) * Dh]
                sT = lax.dot_general(k_bh, q_bh, (((1,), (0,)), ((), ())),
                                     preferred_element_type=jnp.float32)
                m = jnp.max(sT, axis=0, keepdims=True)
                p = jnp.exp(sT - m)
                l = jnp.sum(p, axis=0, keepdims=True)
                oT = lax.dot_general(v_bh, p.astype(jnp.bfloat16),
                                     (((0,), (0,)), ((), ())),
                                     preferred_element_type=jnp.float32)
                o_acc[j, i, :, :] = oT.astype(jnp.bfloat16)
                ml_acc[j, i, 0:1, :] = m
                ml_acc[j, i, 1:2, :] = l

        def start_exchange(j, step):
            partner = my ^ (1 << ((step + j) % LOG2_N))
            o_rd = pltpu.make_async_remote_copy(
                src_ref=o_acc.at[j],
                dst_ref=o_rx.at[step, j],
                send_sem=o_send_sems.at[step, j],
                recv_sem=o_recv_sems.at[step, j],
                device_id=(partner,),
                device_id_type=pl.DeviceIdType.MESH,
            )
            ml_rd = pltpu.make_async_remote_copy(
                src_ref=ml_acc.at[j],
                dst_ref=ml_rx.at[step, j],
                send_sem=ml_send_sems.at[step, j],
                recv_sem=ml_recv_sems.at[step, j],
                device_id=(partner,),
                device_id_type=pl.DeviceIdType.MESH,
            )
            o_rd.start()
            ml_rd.start()
            return o_rd, ml_rd

        def combine(j, step):
            m1 = ml_acc[j, :, 0:1, :]
            l1 = ml_acc[j, :, 1:2, :]
            m2 = ml_rx[step, j, :, 0:1, :]
            l2 = ml_rx[step, j, :, 1:2, :]
            mn = jnp.maximum(m1, m2)
            a1 = jnp.exp(m1 - mn)
            a2 = jnp.exp(m2 - mn)
            ml_acc[j, :, 0:1, :] = mn
            ml_acc[j, :, 1:2, :] = a1 * l1 + a2 * l2
            o_new = (a1 * o_acc[j].astype(jnp.float32)
                     + a2 * o_rx[step, j].astype(jnp.float32))
            o_acc[j] = o_new.astype(jnp.bfloat16)

        wo = wo_ref[...].astype(jnp.bfloat16)

        def project(j):
            b, hg = divmod(j, 2)
            linv = 1.0 / ml_acc[j, :, 1:2, :]
            acc = jnp.zeros((Sq, D), jnp.float32)
            for i in range(SB):
                h = hg * SB + i
                o_n = (o_acc[j, i, :, :].astype(jnp.float32)
                       * linv[i, :, :]).astype(jnp.bfloat16)
                wo_h = wo[h * Dh:(h + 1) * Dh, :]
                acc = acc + lax.dot_general(
                    o_n, wo_h, (((0,), (0,)), ((), ())),
                    preferred_element_type=jnp.float32)
            if hg == 0:
                out_ref[b, :, :] = acc
            else:
                out_ref[b, :, :] = out_ref[b, :, :] + acc

        rds = [None] * S
        for j in range(S):
            flash_strip(j)
            rds[j] = start_exchange(j, 0)
        for step in range(LOG2_N):
            for j in range(S):
                rds[j][0].wait()
                rds[j][1].wait()
                combine(j, step)
                if step + 1 < LOG2_N:
                    rds[j] = start_exchange(j, step + 1)
                else:
                    project(j)

    return pl.pallas_call(
        body,
        out_shape=jax.ShapeDtypeStruct((B, Sq, D), jnp.float32),
        in_specs=[pl.BlockSpec(memory_space=pltpu.VMEM)] * 5,
        out_specs=pl.BlockSpec(memory_space=pltpu.VMEM),
        scratch_shapes=[
            pltpu.VMEM((S, SB, Dh, Sq), jnp.bfloat16),
            pltpu.VMEM((S, SB, 2, Sq), jnp.float32),
            pltpu.VMEM((LOG2_N, S, SB, Dh, Sq), jnp.bfloat16),
            pltpu.VMEM((LOG2_N, S, SB, 2, Sq), jnp.float32),
            pltpu.SemaphoreType.DMA((LOG2_N, S)),
            pltpu.SemaphoreType.DMA((LOG2_N, S)),
            pltpu.SemaphoreType.DMA((LOG2_N, S)),
            pltpu.SemaphoreType.DMA((LOG2_N, S)),
        ],
    )(x, Wq, Wo, K_ext, V_ext)


# device time: 50698 ns/iter; 1.3726x vs baseline; 1.0101x over previous
import jax
import jax.numpy as jnp
from jax import lax
from jax.experimental import pallas as pl
from jax.experimental.pallas import tpu as pltpu

N_DEV = 32
LOG2_N = 5
B, Sq, D = 2, 256, 768
Hq, Dh = 8, 64
HD = Hq * Dh
BH = B * Hq
S = 4
SB = BH // S


def kernel(x, Wq, Wo, K_ext, V_ext):
    skv_loc = K_ext.shape[1]

    def body(x_ref, wq_ref, wo_ref, k_ref, v_ref, out_ref,
             o_acc, ml_acc, o_rx, ml_rx,
             o_send_sems, o_recv_sems, ml_send_sems, ml_recv_sems):
        my = lax.axis_index("i")

        x2 = x_ref[...].reshape(B * Sq, D).astype(jnp.bfloat16)
        wq = wq_ref[...].astype(jnp.bfloat16)

        kbs = []
        vbs = []
        for b in range(B):
            kbs.append(k_ref[b, :, :, :].reshape(skv_loc, HD).astype(jnp.bfloat16))
            vbs.append(v_ref[b, :, :, :].reshape(skv_loc, HD).astype(jnp.bfloat16))

        def flash_strip(j):
            sb, hg = divmod(j, 2)
            qTs = lax.dot_general(
                wq[:, hg * SB * Dh:(hg + 1) * SB * Dh],
                x2[sb * Sq:(sb + 1) * Sq, :],
                (((0,), (1,)), ((), ())),
                preferred_element_type=jnp.float32)
            qTs = (qTs * 0.125).astype(jnp.bfloat16)
            for i in range(SB):
                bh = j * SB + i
                b, h = divmod(bh, Hq)
                q_bh = qTs[i * Dh:(i + 1) * Dh, :]
                k_bh = kbs[b][:, h * Dh:(h + 1) * Dh]
                v_bh = vbs[b][:, h * Dh:(h + 1) * Dh]
                sT = lax.dot_general(k_bh, q_bh, (((1,), (0,)), ((), ())),
                                     preferred_element_type=jnp.float32)
                m = jnp.max(sT, axis=0, keepdims=True)
                p = jnp.exp(sT - m)
                l = jnp.sum(p, axis=0, keepdims=True)
                oT = lax.dot_general(v_bh, p.astype(jnp.bfloat16),
                                     (((0,), (0,)), ((), ())),
                                     preferred_element_type=jnp.float32)
                o_acc[j, i, :, :] = oT.astype(jnp.bfloat16)
                ml_acc[j, i, 0:1, :] = m
                ml_acc[j, i, 1:2, :] = l

        def start_exchange(j, step):
            partner = my ^ (1 << ((step + j) % LOG2_N))
            o_rd = pltpu.make_async_remote_copy(
                src_ref=o_acc.at[j],
                dst_ref=o_rx.at[step, j],
                send_sem=o_send_sems.at[step, j],
                recv_sem=o_recv_sems.at[step, j],
                device_id=(partner,),
                device_id_type=pl.DeviceIdType.MESH,
            )
            ml_rd = pltpu.make_async_remote_copy(
                src_ref=ml_acc.at[j],
                dst_ref=ml_rx.at[step, j],
                send_sem=ml_send_sems.at[step, j],
                recv_sem=ml_recv_sems.at[step, j],
                device_id=(partner,),
                device_id_type=pl.DeviceIdType.MESH,
            )
            o_rd.start()
            ml_rd.start()
            return o_rd, ml_rd

        def combine(j, step):
            m1 = ml_acc[j, :, 0:1, :]
            l1 = ml_acc[j, :, 1:2, :]
            m2 = ml_rx[step, j, :, 0:1, :]
            l2 = ml_rx[step, j, :, 1:2, :]
            mn = jnp.maximum(m1, m2)
            a1 = jnp.exp(m1 - mn)
            a2 = jnp.exp(m2 - mn)
            ml_acc[j, :, 0:1, :] = mn
            ml_acc[j, :, 1:2, :] = a1 * l1 + a2 * l2
            o_new = (a1 * o_acc[j].astype(jnp.float32)
                     + a2 * o_rx[step, j].astype(jnp.float32))
            o_acc[j] = o_new.astype(jnp.bfloat16)

        wo = wo_ref[...].astype(jnp.bfloat16)

        def project(j):
            b, hg = divmod(j, 2)
            linv = 1.0 / ml_acc[j, :, 1:2, :]
            acc = jnp.zeros((Sq, D), jnp.float32)
            for i in range(SB):
                h = hg * SB + i
                o_n = (o_acc[j, i, :, :].astype(jnp.float32)
                       * linv[i, :, :]).astype(jnp.bfloat16)
                wo_h = wo[h * Dh:(h + 1) * Dh, :]
                acc = acc + lax.dot_general(
                    o_n, wo_h, (((0,), (0,)), ((), ())),
                    preferred_element_type=jnp.float32)
            if hg == 0:
                out_ref[b, :, :] = acc
            else:
                out_ref[b, :, :] = out_ref[b, :, :] + acc

        rds = [None] * S
        for j in range(S):
            flash_strip(j)
            rds[j] = start_exchange(j, 0)
        for step in range(LOG2_N):
            for j in range(S):
                rds[j][0].wait()
                rds[j][1].wait()
                combine(j, step)
                if step + 1 < LOG2_N:
                    rds[j] = start_exchange(j, step + 1)
                else:
                    project(j)

    return pl.pallas_call(
        body,
        out_shape=jax.ShapeDtypeStruct((B, Sq, D), jnp.float32),
        in_specs=[pl.BlockSpec(memory_space=pltpu.VMEM)] * 5,
        out_specs=pl.BlockSpec(memory_space=pltpu.VMEM),
        scratch_shapes=[
            pltpu.VMEM((S, SB, Dh, Sq), jnp.bfloat16),
            pltpu.VMEM((S, SB, 2, Sq), jnp.float32),
            pltpu.VMEM((LOG2_N, S, SB, Dh, Sq), jnp.bfloat16),
            pltpu.VMEM((LOG2_N, S, SB, 2, Sq), jnp.float32),
            pltpu.SemaphoreType.DMA((LOG2_N, S)),
            pltpu.SemaphoreType.DMA((LOG2_N, S)),
            pltpu.SemaphoreType.DMA((LOG2_N, S)),
            pltpu.SemaphoreType.DMA((LOG2_N, S)),
        ],
    )(x, Wq, Wo, K_ext, V_ext)
